# initial kernel scaffold (unmeasured)
import jax
import jax.numpy as jnp
from jax import lax
from jax.experimental import pallas as pl
from jax.experimental.pallas import tpu as pltpu


def kernel(
    x,
):
    def body(*refs):
        pass

    out_shape = jax.ShapeDtypeStruct(..., jnp.float32)
    return pl.pallas_call(body, out_shape=out_shape)(...)


# baseline (device time: 742129 ns/iter reference)
import jax
import jax.numpy as jnp
from jax import lax
from jax.experimental import pallas as pl
from jax.experimental.pallas import tpu as pltpu

N_DEV = 8


def kernel(x):
    m_per, n = x.shape
    half = m_per // 2

    def body(x_ref, out_ref, local_sem, cw_send, cw_recv, ccw_send, ccw_recv):
        me = lax.axis_index("i")
        left = lax.rem(me + N_DEV - 1, N_DEV)
        right = lax.rem(me + 1, N_DEV)

        barrier_sem = pltpu.get_barrier_semaphore()
        for nbr in (left, right):
            pl.semaphore_signal(
                barrier_sem, inc=1,
                device_id=(nbr,), device_id_type=pl.DeviceIdType.MESH,
            )
        pl.semaphore_wait(barrier_sem, 2)

        local_copy = pltpu.make_async_copy(
            x_ref, out_ref.at[pl.ds(me * m_per, m_per)], local_sem
        )
        local_copy.start()

        for h in range(N_DEV - 1):
            oc = lax.rem(me + N_DEV - h, N_DEV)
            cw_src = x_ref.at[pl.ds(0, half)] if h == 0 else (
                out_ref.at[pl.ds(oc * m_per, half)]
            )
            cw = pltpu.make_async_remote_copy(
                src_ref=cw_src,
                dst_ref=out_ref.at[pl.ds(oc * m_per, half)],
                send_sem=cw_send.at[h],
                recv_sem=cw_recv.at[h],
                device_id=(right,),
                device_id_type=pl.DeviceIdType.MESH,
            )
            occ = lax.rem(me + h, N_DEV)
            ccw_src = x_ref.at[pl.ds(half, half)] if h == 0 else (
                out_ref.at[pl.ds(occ * m_per + half, half)]
            )
            ccw = pltpu.make_async_remote_copy(
                src_ref=ccw_src,
                dst_ref=out_ref.at[pl.ds(occ * m_per + half, half)],
                send_sem=ccw_send.at[h],
                recv_sem=ccw_recv.at[h],
                device_id=(left,),
                device_id_type=pl.DeviceIdType.MESH,
            )
            cw.start()
            ccw.start()
            cw.wait()
            ccw.wait()

        local_copy.wait()

    return pl.pallas_call(
        body,
        out_shape=jax.ShapeDtypeStruct((N_DEV * m_per, n), x.dtype),
        in_specs=[pl.BlockSpec(memory_space=pl.ANY)],
        out_specs=pl.BlockSpec(memory_space=pl.ANY),
        scratch_shapes=[
            pltpu.SemaphoreType.DMA,
            pltpu.SemaphoreType.DMA((N_DEV - 1,)),
            pltpu.SemaphoreType.DMA((N_DEV - 1,)),
            pltpu.SemaphoreType.DMA((N_DEV - 1,)),
            pltpu.SemaphoreType.DMA((N_DEV - 1,)),
        ],
        compiler_params=pltpu.CompilerParams(collective_id=0),
    )(x)


# device time: 729326 ns/iter; 1.0176x vs baseline; 1.0176x over previous
import jax
import jax.numpy as jnp
from jax import lax
from jax.experimental import pallas as pl
from jax.experimental.pallas import tpu as pltpu

N_DEV = 8
N_HOP = N_DEV - 1
SEG = 2


def kernel(x):
    m_per, n = x.shape
    half = m_per // 2
    seg = half // SEG

    def body(x_ref, out_ref, local_sem, cw_send, cw_recv, ccw_send, ccw_recv):
        me = lax.axis_index("i")
        left = lax.rem(me + N_DEV - 1, N_DEV)
        right = lax.rem(me + 1, N_DEV)

        barrier_sem = pltpu.get_barrier_semaphore()
        for nbr in (left, right):
            pl.semaphore_signal(
                barrier_sem, inc=1,
                device_id=(nbr,), device_id_type=pl.DeviceIdType.MESH,
            )
        pl.semaphore_wait(barrier_sem, 2)

        local_copy = pltpu.make_async_copy(
            x_ref, out_ref.at[pl.ds(me * m_per, m_per)], local_sem
        )
        local_copy.start()

        def reg_cw(o, s):
            return out_ref.at[pl.ds(o * m_per + s * seg, seg)]

        def reg_ccw(o, s):
            return out_ref.at[pl.ds(o * m_per + half + s * seg, seg)]

        def rdma(src, dst, ssem, rsem, dev):
            return pltpu.make_async_remote_copy(
                src_ref=src, dst_ref=dst, send_sem=ssem, recv_sem=rsem,
                device_id=(dev,), device_id_type=pl.DeviceIdType.MESH,
            )

        for h in range(N_HOP):
            oc = lax.rem(me + N_DEV - h, N_DEV)
            occ = lax.rem(me + h, N_DEV)
            for s in range(SEG):
                if h > 0:
                    rdma(reg_cw(oc, s), reg_cw(oc, s),
                         cw_send.at[h - 1, s], cw_recv.at[h - 1, s],
                         right).wait_recv()
                cw_src = x_ref.at[pl.ds(s * seg, seg)] if h == 0 else (
                    reg_cw(oc, s)
                )
                rdma(cw_src, reg_cw(oc, s),
                     cw_send.at[h, s], cw_recv.at[h, s], right).start()

                if h > 0:
                    rdma(reg_ccw(occ, s), reg_ccw(occ, s),
                         ccw_send.at[h - 1, s], ccw_recv.at[h - 1, s],
                         left).wait_recv()
                ccw_src = x_ref.at[pl.ds(half + s * seg, seg)] if h == 0 else (
                    reg_ccw(occ, s)
                )
                rdma(ccw_src, reg_ccw(occ, s),
                     ccw_send.at[h, s], ccw_recv.at[h, s], left).start()

        oc_last = lax.rem(me + 1, N_DEV)
        occ_last = lax.rem(me + N_DEV - 1, N_DEV)
        for s in range(SEG):
            rdma(reg_cw(oc_last, s), reg_cw(oc_last, s),
                 cw_send.at[N_HOP - 1, s], cw_recv.at[N_HOP - 1, s],
                 right).wait_recv()
            rdma(reg_ccw(occ_last, s), reg_ccw(occ_last, s),
                 ccw_send.at[N_HOP - 1, s], ccw_recv.at[N_HOP - 1, s],
                 left).wait_recv()
        for h in range(N_HOP):
            for s in range(SEG):
                rdma(reg_cw(0, s), reg_cw(0, s),
                     cw_send.at[h, s], cw_recv.at[h, s], right).wait_send()
                rdma(reg_ccw(0, s), reg_ccw(0, s),
                     ccw_send.at[h, s], ccw_recv.at[h, s], left).wait_send()

        local_copy.wait()

    return pl.pallas_call(
        body,
        out_shape=jax.ShapeDtypeStruct((N_DEV * m_per, n), x.dtype),
        in_specs=[pl.BlockSpec(memory_space=pl.ANY)],
        out_specs=pl.BlockSpec(memory_space=pl.ANY),
        scratch_shapes=[
            pltpu.SemaphoreType.DMA,
            pltpu.SemaphoreType.DMA((N_HOP, SEG)),
            pltpu.SemaphoreType.DMA((N_HOP, SEG)),
            pltpu.SemaphoreType.DMA((N_HOP, SEG)),
            pltpu.SemaphoreType.DMA((N_HOP, SEG)),
        ],
        compiler_params=pltpu.CompilerParams(collective_id=0),
    )(x)


# device time: 531585 ns/iter; 1.3961x vs baseline; 1.3720x over previous
import jax
import jax.numpy as jnp
from jax import lax
from jax.experimental import pallas as pl
from jax.experimental.pallas import tpu as pltpu

N_DEV = 8
N_HOP = 7

MOVES = (
    (1, 2, 1, 4, 1, 2, 1),
    (2, 4, 2, 1, 2, 4, 2),
    (4, 1, 4, 2, 4, 1, 4),
)
PREFIX = tuple(
    tuple(__import__("functools").reduce(lambda a, b: a ^ b, mv[: k + 1]) for k in range(N_HOP))
    for mv in MOVES
)


def kernel(x):
    m_per, n = x.shape
    rows = (1368, 1368, m_per - 2 * 1368)
    offs = (0, 1368, 2 * 1368)

    def body(x_ref, out_ref, local_sem, send_sems, recv_sems):
        me = lax.axis_index("i")

        def g(i):
            return i ^ ((i >> 1) & 1)

        cme = g(me)

        barrier_sem = pltpu.get_barrier_semaphore()
        for axis_mask in (1, 2, 4):
            pl.semaphore_signal(
                barrier_sem, inc=1,
                device_id=(g(cme ^ axis_mask),),
                device_id_type=pl.DeviceIdType.MESH,
            )
        pl.semaphore_wait(barrier_sem, 3)

        local_copy = pltpu.make_async_copy(
            x_ref, out_ref.at[pl.ds(me * m_per, m_per)], local_sem
        )
        local_copy.start()

        def region(o, r):
            return out_ref.at[pl.ds(o * m_per + offs[r], rows[r])]

        def rdma(src, dst, ssem, rsem, dev):
            return pltpu.make_async_remote_copy(
                src_ref=src, dst_ref=dst, send_sem=ssem, recv_sem=rsem,
                device_id=(dev,), device_id_type=pl.DeviceIdType.MESH,
            )

        for h in range(N_HOP):
            for r in range(3):
                o_recv_prev = g(cme ^ PREFIX[r][h - 1]) if h > 0 else None
                if h > 0:
                    rdma(region(o_recv_prev, r), region(o_recv_prev, r),
                         send_sems.at[r, h - 1], recv_sems.at[r, h - 1],
                         me).wait_recv()
                src = x_ref.at[pl.ds(offs[r], rows[r])] if h == 0 else (
                    region(o_recv_prev, r)
                )
                o_send = me if h == 0 else o_recv_prev
                partner = g(cme ^ MOVES[r][h])
                rdma(src, region(o_send, r),
                     send_sems.at[r, h], recv_sems.at[r, h], partner).start()

        for r in range(3):
            o_last = g(cme ^ PREFIX[r][N_HOP - 1])
            rdma(region(o_last, r), region(o_last, r),
                 send_sems.at[r, N_HOP - 1], recv_sems.at[r, N_HOP - 1],
                 me).wait_recv()
        for h in range(N_HOP):
            for r in range(3):
                rdma(x_ref.at[pl.ds(offs[r], rows[r])], region(me, r),
                     send_sems.at[r, h], recv_sems.at[r, h], me).wait_send()

        local_copy.wait()

    return pl.pallas_call(
        body,
        out_shape=jax.ShapeDtypeStruct((N_DEV * m_per, n), x.dtype),
        in_specs=[pl.BlockSpec(memory_space=pl.ANY)],
        out_specs=pl.BlockSpec(memory_space=pl.ANY),
        scratch_shapes=[
            pltpu.SemaphoreType.DMA,
            pltpu.SemaphoreType.DMA((3, N_HOP)),
            pltpu.SemaphoreType.DMA((3, N_HOP)),
        ],
        compiler_params=pltpu.CompilerParams(collective_id=0),
    )(x)


# device time: 527780 ns/iter; 1.4061x vs baseline; 1.0072x over previous
import jax
import jax.numpy as jnp
from jax import lax
from jax.experimental import pallas as pl
from jax.experimental.pallas import tpu as pltpu

N_DEV = 8
N_HOP = 7

MOVES = (
    (1, 2, 1, 4, 1, 2, 1),
    (2, 4, 2, 1, 2, 4, 2),
    (4, 1, 4, 2, 4, 1, 4),
)
PREFIX = tuple(
    tuple(__import__("functools").reduce(lambda a, b: a ^ b, mv[: k + 1]) for k in range(N_HOP))
    for mv in MOVES
)


SEG = 2


def kernel(x):
    m_per, n = x.shape
    rows = (1368, 1368, m_per - 2 * 1368)
    offs = (0, 1368, 2 * 1368)
    segrows = tuple((rw - rw // 2 // 8 * 8, rw // 2 // 8 * 8) for rw in rows)
    segoffs = tuple((0, sr[0]) for sr in segrows)

    def body(x_ref, out_ref, local_sem, send_sems, recv_sems):
        me = lax.axis_index("i")

        def g(i):
            return i ^ ((i >> 1) & 1)

        cme = g(me)

        barrier_sem = pltpu.get_barrier_semaphore()
        for axis_mask in (1, 2, 4):
            pl.semaphore_signal(
                barrier_sem, inc=1,
                device_id=(g(cme ^ axis_mask),),
                device_id_type=pl.DeviceIdType.MESH,
            )
        pl.semaphore_wait(barrier_sem, 3)

        local_copy = pltpu.make_async_copy(
            x_ref, out_ref.at[pl.ds(me * m_per, m_per)], local_sem
        )
        local_copy.start()

        def region(o, r, s):
            return out_ref.at[
                pl.ds(o * m_per + offs[r] + segoffs[r][s], segrows[r][s])
            ]

        def xseg(r, s):
            return x_ref.at[pl.ds(offs[r] + segoffs[r][s], segrows[r][s])]

        def rdma(src, dst, ssem, rsem, dev):
            return pltpu.make_async_remote_copy(
                src_ref=src, dst_ref=dst, send_sem=ssem, recv_sem=rsem,
                device_id=(dev,), device_id_type=pl.DeviceIdType.MESH,
            )

        for h in range(N_HOP):
            for r in range(3):
                o_prev = g(cme ^ PREFIX[r][h - 1]) if h > 0 else None
                partner = g(cme ^ MOVES[r][h])
                for s in range(SEG):
                    if h > 0:
                        rdma(region(o_prev, r, s), region(o_prev, r, s),
                             send_sems.at[r, h - 1, s],
                             recv_sems.at[r, h - 1, s], me).wait_recv()
                    src = xseg(r, s) if h == 0 else region(o_prev, r, s)
                    o_send = me if h == 0 else o_prev
                    rdma(src, region(o_send, r, s),
                         send_sems.at[r, h, s], recv_sems.at[r, h, s],
                         partner).start()

        for r in range(3):
            o_last = g(cme ^ PREFIX[r][N_HOP - 1])
            for s in range(SEG):
                rdma(region(o_last, r, s), region(o_last, r, s),
                     send_sems.at[r, N_HOP - 1, s],
                     recv_sems.at[r, N_HOP - 1, s], me).wait_recv()
        for h in range(N_HOP):
            for r in range(3):
                for s in range(SEG):
                    rdma(xseg(r, s), region(me, r, s),
                         send_sems.at[r, h, s], recv_sems.at[r, h, s],
                         me).wait_send()

        local_copy.wait()

    return pl.pallas_call(
        body,
        out_shape=jax.ShapeDtypeStruct((N_DEV * m_per, n), x.dtype),
        in_specs=[pl.BlockSpec(memory_space=pl.ANY)],
        out_specs=pl.BlockSpec(memory_space=pl.ANY),
        scratch_shapes=[
            pltpu.SemaphoreType.DMA,
            pltpu.SemaphoreType.DMA((3, N_HOP, SEG)),
            pltpu.SemaphoreType.DMA((3, N_HOP, SEG)),
        ],
        compiler_params=pltpu.CompilerParams(collective_id=0),
    )(x)
